# baseline (speedup 1.0000x reference)
import functools

import numpy as np
import jax
import jax.numpy as jnp
from jax import lax
from jax.experimental import pallas as pl
from jax.experimental.pallas import tpu as pltpu
from jax.experimental.pallas import tpu_sc as plsc

NENT = 50000
NREL = 237
D = 256
T = 4
TD = 64
K = 64
B = 512
NEG = 256

NENT_PAD = 50048
ET = 2944
NT = 17
NVR = NENT_PAD // 16
PAD_VAL = -1e30

NC = 2
NS = 16
NW = NC * NS
RPW = B // NW
L = 16

CAP = 128
NEG_BIG = -3.0e38

_KLO = int(np.int32(np.float32(-1e28).view(np.int32)) ^ np.int32(0x7FFFFFFF))

_SC_PARAMS = pltpu.CompilerParams(needs_layout_passes=False)


def _lanes():
    return lax.broadcasted_iota(jnp.int32, (L,), 0)


def _bc_f(x):
    return jnp.full((L,), x, jnp.float32)


def _bc_i(x):
    return jnp.full((L,), x, jnp.int32)


def _lane_f(vec, i):
    return jnp.sum(jnp.where(_lanes() == i, vec, jnp.float32(0.0)))


def _lane_i(vec, i):
    return jnp.sum(jnp.where(_lanes() == i, vec, jnp.int32(0)))


def _key_vec(x):
    b = plsc.bitcast(x + 0.0, jnp.int32)
    return jnp.where(b >= 0, b, b ^ jnp.int32(0x7FFFFFFF))


def _unmap_scalar(k):
    kv = _bc_i(k)
    bv = jnp.where(kv >= 0, kv, kv ^ jnp.int32(0x7FFFFFFF))
    return jnp.max(plsc.bitcast(bv, jnp.float32))


def _avg_i32(a, b):
    return (a >> 1) + (b >> 1) + (a & b & 1)


def _gather_body(pos0_hbm, pos1_hbm, ent_hbm, rel_hbm, te2_hbm, trl_hbm,
                 wrelT_hbm, hr_out, hrt_out, wrel_out,
                 idx0_v, idx1_v, tidx4_v, th2_v, h_v, r_v, a_v, b_v,
                 wt_v, wtmp_v, sem):
    wid = lax.axis_index("s") * NC + lax.axis_index("c")
    base = wid * RPW

    pltpu.sync_copy(pos0_hbm.at[pl.ds(base, RPW)], idx0_v)
    pltpu.sync_copy(pos1_hbm.at[pl.ds(base, RPW)], idx1_v)

    pltpu.async_copy(ent_hbm.at[idx0_v], h_v, sem).wait()
    pltpu.async_copy(rel_hbm.at[idx1_v], r_v, sem).wait()
    for i in range(RPW):
        for j in range(D // L):
            sl = pl.ds(j * L, L)
            h_v[i, sl] = h_v[i, sl] * r_v[i, sl]
    pltpu.sync_copy(h_v, hr_out.at[pl.ds(base, RPW)])

    i0 = idx0_v[...]
    i1 = idx1_v[...]
    pltpu.async_copy(trl_hbm.at[idx1_v], b_v, sem).wait()
    for t in range(T):
        tidx4_v[t, :] = (i0 >> 1) + t * (NENT // 2)
    cps = [pltpu.async_copy(te2_hbm.at[tidx4_v.at[t]], th2_v.at[t], sem)
           for t in range(T)]
    for cp in cps:
        cp.wait()
    pe = i0 & 1
    for i in range(RPW):
        p_i = _lane_i(pe, i)
        odd = p_i == 1
        for t in range(T):
            for j in range(TD // L):
                lo = th2_v[t, i, pl.ds(j * L, L)]
                hi = th2_v[t, i, pl.ds(TD + j * L, L)]
                th = jnp.where(odd, hi, lo)
                sl = pl.ds(t * TD + j * L, L)
                a_v[i, sl] = th * b_v[i, sl]
    pltpu.sync_copy(a_v, hrt_out.at[pl.ds(base, RPW)])

    pltpu.sync_copy(wrelT_hbm, wt_v)
    for t in range(T):
        wtmp_v[pl.ds(t * L, L)] = plsc.load_gather(wt_v, [i1 + t * 240])
    for t in range(T):
        pltpu.sync_copy(wtmp_v.at[pl.ds(t * L, L)],
                        wrel_out.at[t, pl.ds(base, RPW)])


def _run_gather(pos0, pos1, ent_emb, rel_emb, te_flat, trl_flat, wrelT_pad):
    mesh = plsc.VectorSubcoreMesh(core_axis_name="c", subcore_axis_name="s")
    f = pl.kernel(
        _gather_body,
        out_type=(
            jax.ShapeDtypeStruct((B, D), jnp.float32),
            jax.ShapeDtypeStruct((B, T * TD), jnp.float32),
            jax.ShapeDtypeStruct((T, B), jnp.float32),
        ),
        mesh=mesh,
        scratch_types=[
            pltpu.VMEM((RPW,), jnp.int32),
            pltpu.VMEM((RPW,), jnp.int32),
            pltpu.VMEM((T, RPW), jnp.int32),
            pltpu.VMEM((T, RPW, 2 * TD), jnp.float32),
            pltpu.VMEM((RPW, D), jnp.float32),
            pltpu.VMEM((RPW, D), jnp.float32),
            pltpu.VMEM((RPW, T * TD), jnp.float32),
            pltpu.VMEM((RPW, T * TD), jnp.float32),
            pltpu.VMEM((T * 240,), jnp.float32),
            pltpu.VMEM((T * L,), jnp.float32),
            pltpu.SemaphoreType.DMA,
        ],
        compiler_params=_SC_PARAMS,
        name="mulde_gather_sc",
    )
    return f(pos0, pos1, ent_emb, rel_emb, te_flat, trl_flat, wrelT_pad)


def _mm_body(hr_ref, ent_ref, score_ref, mx_ref, sm_ref, sq_ref):
    i = pl.program_id(0)
    s = lax.dot_general(hr_ref[...], ent_ref[...],
                        (((1,), (1,)), ((), ())),
                        preferred_element_type=jnp.float32)
    col = i * ET + lax.broadcasted_iota(jnp.int32, (1, ET), 1)
    valid = col < NENT
    s = jnp.where(valid, s, PAD_VAL)
    score_ref[...] = s
    s_z = jnp.where(valid, s, 0.0)
    mx_ref[0, 0, :] = jnp.max(s, axis=1)
    sm_ref[0, 0, :] = jnp.sum(s_z, axis=1)
    sq_ref[0, 0, :] = jnp.sum(s_z * s_z, axis=1)


def _run_matmul(hr, ent_pad):
    return pl.pallas_call(
        _mm_body,
        grid=(NT,),
        in_specs=[
            pl.BlockSpec((B, D), lambda i: (0, 0)),
            pl.BlockSpec((ET, D), lambda i: (i, 0)),
        ],
        out_specs=[
            pl.BlockSpec((B, ET), lambda i: (0, i)),
            pl.BlockSpec((1, 1, B), lambda i: (i, 0, 0)),
            pl.BlockSpec((1, 1, B), lambda i: (i, 0, 0)),
            pl.BlockSpec((1, 1, B), lambda i: (i, 0, 0)),
        ],
        out_shape=[
            jax.ShapeDtypeStruct((B, NENT_PAD), jnp.float32),
            jax.ShapeDtypeStruct((NT, 1, B), jnp.float32),
            jax.ShapeDtypeStruct((NT, 1, B), jnp.float32),
            jax.ShapeDtypeStruct((NT, 1, B), jnp.float32),
        ],
        name="mulde_score_mm",
    )(hr, ent_pad)


UNR = 8
CBUF = CAP + L
BIGI = 1 << 30
NEC = K
EQB = L * NEC


def _count_row_acc(row_v, tau):
    tb = _bc_f(tau)

    def body(i, carry):
        a0, a1 = carry
        for u in range(UNR):
            x = row_v[pl.ds((i * UNR + u) * L, L)]
            c = jnp.where(x > tb, 1, 0)
            if u % 2 == 0:
                a0 = a0 + c
            else:
                a1 = a1 + c
        return a0, a1

    z = jnp.zeros((L,), jnp.int32)
    a0, a1 = lax.fori_loop(0, NVR // UNR, body, (z, z))
    return a0 + a1


def _count_row(row_v, tau):
    return jnp.sum(_count_row_acc(row_v, tau))


def _collect_row(row_v, tau, acc, cidx_v):
    tb = _bc_f(tau)
    offs = plsc.cumsum(acc) - acc

    def body(i, st):
        pos, gid = st
        for u in range(UNR):
            x = row_v[pl.ds((i * UNR + u) * L, L)]
            m = x > tb
            plsc.store_scatter(cidx_v, [pos], gid, mask=m)
            pos = pos + jnp.where(m, 1, 0)
            gid = gid + L
        return pos, gid

    lax.fori_loop(0, NVR // UNR, body, (offs, _lanes()))


def _topk_body(score_hbm, tau_hbm, mx_hbm, pos2_hbm, neg_hbm,
               sval_out, sidx_out, pos_out, neg_out,
               rowA_v, rowB_v, tau_v, mx_v, p2_v, cidx_v, eqi_v,
               fval_v, fidx_v, negi_v, nego_v, pv_v, semA, semB):
    wid = lax.axis_index("s") * NC + lax.axis_index("c")
    base = wid * RPW
    lanes = _lanes()
    lane0 = lanes == 0
    nbv = _bc_f(jnp.float32(NEG_BIG))
    bigv = _bc_i(jnp.int32(BIGI))

    pltpu.sync_copy(tau_hbm.at[pl.ds(base, RPW)], tau_v)
    pltpu.sync_copy(mx_hbm.at[pl.ds(base, RPW)], mx_v)
    pltpu.sync_copy(pos2_hbm.at[pl.ds(base, RPW)], p2_v)

    def process(row_v, rl, posvec):
        row = base + rl
        tau0 = _lane_f(tau_v[...], rl)
        mxs = _lane_f(mx_v[...], rl)
        k_mx = jnp.max(_key_vec(_bc_f(mxs)))

        acc0 = _count_row_acc(row_v, tau0)
        cnt0 = jnp.sum(acc0)
        ok0 = (cnt0 >= K) & (cnt0 <= CAP)

        def needs_fix():
            k_tau = jnp.max(_key_vec(_bc_f(tau0)))
            lo0 = jnp.where(cnt0 < K, jnp.int32(_KLO), k_tau)
            hi0 = jnp.where(cnt0 < K, k_tau, k_mx)
            ch0 = jnp.where(cnt0 < K, cnt0, jnp.int32(0))

            def cond(st):
                klo, khi, cnt_hi, acc_tau, accepted = st
                return (accepted == 0) & (_avg_i32(klo, khi) != klo)

            def body(st):
                klo, khi, cnt_hi, acc_tau, accepted = st
                kmid = _avg_i32(klo, khi)
                tm = _unmap_scalar(kmid)
                c = _count_row(row_v, tm)
                acc = (c >= K) & (c <= CAP)
                ge = c >= K
                return (jnp.where(ge, kmid, klo),
                        jnp.where(ge, khi, kmid),
                        jnp.where(ge, cnt_hi, c),
                        jnp.where(acc, tm, acc_tau),
                        jnp.where(acc, jnp.int32(1), jnp.int32(0)))

            klo, khi, cnt_hi, acc_tau, accepted = lax.while_loop(
                cond, body,
                (lo0, hi0, ch0, jnp.float32(0), jnp.int32(0)))
            return accepted, acc_tau, khi, cnt_hi

        def no_fix():
            return (jnp.int32(1), tau0, jnp.int32(0), jnp.int32(0))

        accepted, tau_f, khi_dg, cgt_dg = lax.cond(ok0, no_fix, needs_fix)

        def normal():
            accf = lax.cond(ok0, lambda: acc0,
                            lambda: _count_row_acc(row_v, tau_f))
            _collect_row(row_v, tau_f, accf, cidx_v)
            return jnp.sum(accf)

        def degen():
            v64 = _unmap_scalar(khi_dg)
            vb = _bc_f(v64)
            c_gt = cgt_dg
            accg = _count_row_acc(row_v, v64)
            offs_g = plsc.cumsum(accg) - accg
            for q in range(EQB // L):
                eqi_v[pl.ds(q * L, L)] = bigv
            lim = lanes * NEC + NEC

            def bodyB(i, st):
                posg, pose, gid = st
                for u in range(UNR):
                    x = row_v[pl.ds((i * UNR + u) * L, L)]
                    mg = x > vb
                    plsc.store_scatter(cidx_v, [posg], gid, mask=mg)
                    posg = posg + jnp.where(mg, 1, 0)
                    me = x == vb
                    me2 = me & (pose < lim)
                    plsc.store_scatter(eqi_v, [pose], gid, mask=me2)
                    pose = pose + jnp.where(me, 1, 0)
                    gid = gid + L
                return posg, pose, gid

            lax.fori_loop(0, NVR // UNR, bodyB,
                          (offs_g, lanes * NEC, lanes))

            def sel(j, _):
                def mn_body(q, mv):
                    return jnp.minimum(mv, eqi_v[pl.ds(q * L, L)])

                mn = jnp.min(lax.fori_loop(0, EQB // L, mn_body, bigv))
                mnb = _bc_i(mn)

                def cl_body(q, _):
                    e = eqi_v[pl.ds(q * L, L)]
                    eqi_v[pl.ds(q * L, L)] = jnp.where(e == mnb, bigv, e)
                    return 0

                lax.fori_loop(0, EQB // L, cl_body, 0)
                plsc.store_scatter(cidx_v, [_bc_i(c_gt + j)], mnb, mask=lane0)
                return 0

            lax.fori_loop(0, K - c_gt, sel, 0)
            return jnp.int32(K)

        cnt_f = lax.cond(accepted == 1, normal, degen)

        vs = []
        js = []
        for q in range(CBUF // L):
            p = q * L + lanes
            valid = p < cnt_f
            idx = jnp.where(valid, cidx_v[pl.ds(q * L, L)], 0)
            vs.append(jnp.where(valid, plsc.load_gather(row_v, [idx]), nbv))
            js.append(jnp.where(valid, idx, bigv))
        nq = CBUF // L

        def ext_body(k, st):
            vv = list(st[:nq])
            jj = list(st[nq:])
            mx8 = vv[0]
            for q in range(1, nq):
                mx8 = jnp.maximum(mx8, vv[q])
            mv = jnp.max(mx8)
            mb = _bc_f(mv)
            mn8 = jnp.where(vv[0] == mb, jj[0], bigv)
            for q in range(1, nq):
                mn8 = jnp.minimum(mn8, jnp.where(vv[q] == mb, jj[q], bigv))
            iv = jnp.min(mn8)
            ivb = _bc_i(iv)
            plsc.store_scatter(fval_v, [_bc_i(k)], mb, mask=lane0)
            plsc.store_scatter(fidx_v, [_bc_i(k)], ivb, mask=lane0)
            for q in range(nq):
                vv[q] = jnp.where((vv[q] == mb) & (jj[q] == ivb), nbv, vv[q])
            return tuple(vv) + tuple(jj)

        lax.fori_loop(0, K, ext_body, tuple(vs) + tuple(js))

        pltpu.sync_copy(fval_v, sval_out.at[row])
        pltpu.sync_copy(fidx_v, sidx_out.at[row])

        pltpu.sync_copy(neg_hbm.at[row], negi_v)
        for j in range(NEG // L):
            idx = negi_v[pl.ds(j * L, L)]
            nego_v[pl.ds(j * L, L)] = plsc.load_gather(row_v, [idx])
        pltpu.sync_copy(nego_v, neg_out.at[row])
        p2 = _lane_i(p2_v[...], rl)
        pos_s = jnp.max(plsc.load_gather(row_v, [_bc_i(p2)]))
        return jnp.where(lanes == rl, pos_s, posvec)

    def row_body(rl, posvec):
        pltpu.sync_copy(score_hbm.at[base + rl], rowA_v)
        return process(rowA_v, rl, posvec)

    posvec = lax.fori_loop(0, RPW, row_body, jnp.zeros((L,), jnp.float32))
    pv_v[...] = posvec
    pltpu.sync_copy(pv_v, pos_out.at[pl.ds(base, RPW)])


def _run_topk(score, tau0, mx, pos2, negative):
    mesh = plsc.VectorSubcoreMesh(core_axis_name="c", subcore_axis_name="s")
    f = pl.kernel(
        _topk_body,
        out_type=(
            jax.ShapeDtypeStruct((B, K), jnp.float32),
            jax.ShapeDtypeStruct((B, K), jnp.int32),
            jax.ShapeDtypeStruct((B,), jnp.float32),
            jax.ShapeDtypeStruct((B, NEG), jnp.float32),
        ),
        mesh=mesh,
        scratch_types=[
            pltpu.VMEM((NENT_PAD,), jnp.float32),
            pltpu.VMEM((NENT_PAD,), jnp.float32),
            pltpu.VMEM((RPW,), jnp.float32),
            pltpu.VMEM((RPW,), jnp.float32),
            pltpu.VMEM((RPW,), jnp.int32),
            pltpu.VMEM((CBUF,), jnp.int32),
            pltpu.VMEM((EQB,), jnp.int32),
            pltpu.VMEM((K,), jnp.float32),
            pltpu.VMEM((K,), jnp.int32),
            pltpu.VMEM((NEG,), jnp.int32),
            pltpu.VMEM((NEG,), jnp.float32),
            pltpu.VMEM((RPW,), jnp.float32),
            pltpu.SemaphoreType.DMA,
            pltpu.SemaphoreType.DMA,
        ],
        compiler_params=_SC_PARAMS,
        name="mulde_topk_sc",
    )
    return f(score, tau0, mx, pos2, negative)


def _log_sigmoid(x):
    return jnp.minimum(x, 0.0) - jnp.log(1.0 + jnp.exp(-jnp.abs(x)))


def _c1_body(pos_ref, neg_ref, a_ref, b_ref):
    a_ref[...] = _log_sigmoid(pos_ref[...])
    b_ref[...] = jnp.mean(_log_sigmoid(-neg_ref[...]), axis=1)


def _run_hardloss(pos_score, neg_score):
    return pl.pallas_call(
        _c1_body,
        out_shape=[
            jax.ShapeDtypeStruct((B,), jnp.float32),
            jax.ShapeDtypeStruct((B,), jnp.float32),
        ],
        name="mulde_hardloss",
    )(pos_score, neg_score)


def kernel(positive, negative, subsampling_weight, epoch, ent_emb, rel_emb,
           Wrel, t_ent_emb, t_rel_emb):
    pos0 = positive[:, 0].astype(jnp.int32)
    pos1 = positive[:, 1].astype(jnp.int32)
    pos2 = positive[:, 2].astype(jnp.int32)
    negative = negative.astype(jnp.int32)
    w = subsampling_weight

    te2 = t_ent_emb.reshape(T * NENT // 2, 2 * TD)
    trl_flat = t_rel_emb.transpose(1, 0, 2).reshape(NREL, T * TD)
    wrelT_pad = jnp.pad(Wrel.T, ((0, 0), (0, 240 - NREL))).reshape(T * 240)
    ent_pad = jnp.pad(ent_emb, ((0, NENT_PAD - NENT), (0, 0)))

    hr, hrt, wrelT = _run_gather(pos0, pos1, ent_emb, rel_emb, te2,
                                 trl_flat, wrelT_pad)

    score, mx_t, sm_t, sq_t = _run_matmul(hr, ent_pad)

    mx = jnp.max(mx_t[:, 0, :], axis=0)
    mu = jnp.sum(sm_t[:, 0, :], axis=0) / NENT
    var = jnp.maximum(jnp.sum(sq_t[:, 0, :], axis=0) / NENT - mu * mu, 0.0)
    tau0 = mu + 2.92 * jnp.sqrt(var)

    s_score, s_idx, pos_score, neg_score = _run_topk(
        score, tau0, mx, pos2, negative)

    a_ls, b_ls = _run_hardloss(pos_score, neg_score)

    te_flat = t_ent_emb.transpose(1, 0, 2).reshape(NENT, T * TD)
    tt_rows = te_flat[s_idx.reshape(-1)]
    hrt4 = hrt.reshape(B, T, TD).transpose(1, 0, 2)
    tt4 = tt_rows.reshape(B, K, T, TD).transpose(2, 0, 1, 3)
    t_score = jnp.sum(hrt4[:, :, None, :] * tt4, axis=-1)

    sw = jnp.sum(w)
    pos_loss = -jnp.sum(a_ls * w) / sw
    neg_loss = -jnp.sum(b_ls * w) / sw
    hard_loss = (pos_loss + neg_loss) / 2.0

    wrel = wrelT.T
    w_soft = jax.nn.softmax(wrel, axis=1)
    agg_t = jnp.einsum('bt,tbk->bk', w_soft, t_score)
    temp = 2.0
    log_p_s = jax.nn.log_softmax(s_score / temp, axis=1)
    p_t = jax.nn.softmax(agg_t / temp, axis=1)
    log_p_t = jax.nn.log_softmax(agg_t / temp, axis=1)
    kl = jnp.sum(p_t * (log_p_t - log_p_s), axis=1)
    soft_loss = jnp.sum(kl * w) / sw
    alpha = jnp.minimum(1.0, jnp.asarray(epoch, jnp.float32) / 10.0)
    loss = hard_loss + alpha * soft_loss
    return (loss, hard_loss, soft_loss)


# speedup vs baseline: 1.1088x; 1.1088x over previous
import functools

import numpy as np
import jax
import jax.numpy as jnp
from jax import lax
from jax.experimental import pallas as pl
from jax.experimental.pallas import tpu as pltpu
from jax.experimental.pallas import tpu_sc as plsc

NENT = 50000
NREL = 237
D = 256
T = 4
TD = 64
K = 64
B = 512
NEG = 256

NENT_PAD = 50048
ET = 2944
NT = 17
NVR = NENT_PAD // 16
PAD_VAL = -1e30

NC = 2
NS = 16
NW = NC * NS
RPW = B // NW
L = 16

CAP = 128
NEG_BIG = -3.0e38

_KLO = int(np.int32(np.float32(-1e28).view(np.int32)) ^ np.int32(0x7FFFFFFF))

_SC_PARAMS = pltpu.CompilerParams(needs_layout_passes=False)


def _lanes():
    return lax.broadcasted_iota(jnp.int32, (L,), 0)


def _bc_f(x):
    return jnp.full((L,), x, jnp.float32)


def _bc_i(x):
    return jnp.full((L,), x, jnp.int32)


def _lane_f(vec, i):
    return jnp.sum(jnp.where(_lanes() == i, vec, jnp.float32(0.0)))


def _lane_i(vec, i):
    return jnp.sum(jnp.where(_lanes() == i, vec, jnp.int32(0)))


def _key_vec(x):
    b = plsc.bitcast(x + 0.0, jnp.int32)
    return jnp.where(b >= 0, b, b ^ jnp.int32(0x7FFFFFFF))


def _unmap_scalar(k):
    kv = _bc_i(k)
    bv = jnp.where(kv >= 0, kv, kv ^ jnp.int32(0x7FFFFFFF))
    return jnp.max(plsc.bitcast(bv, jnp.float32))


def _avg_i32(a, b):
    return (a >> 1) + (b >> 1) + (a & b & 1)


def _gather_body(pos0_hbm, pos1_hbm, ent_hbm, rel_hbm, te_hbm, trl_hbm,
                 wrelT_hbm, hr_out, hrt_out, wrel_out,
                 idx0_v, idx1_v, h_v, r_v, a_v, b_v,
                 wt_v, wtmp_v, sem):
    wid = lax.axis_index("s") * NC + lax.axis_index("c")
    base = wid * RPW

    pltpu.sync_copy(pos0_hbm.at[pl.ds(base, RPW)], idx0_v)
    pltpu.sync_copy(pos1_hbm.at[pl.ds(base, RPW)], idx1_v)

    pltpu.async_copy(ent_hbm.at[idx0_v], h_v, sem).wait()
    pltpu.async_copy(rel_hbm.at[idx1_v], r_v, sem).wait()
    for i in range(RPW):
        for j in range(D // L):
            sl = pl.ds(j * L, L)
            h_v[i, sl] = h_v[i, sl] * r_v[i, sl]
    pltpu.sync_copy(h_v, hr_out.at[pl.ds(base, RPW)])

    i1 = idx1_v[...]
    pltpu.async_copy(te_hbm.at[idx0_v], a_v, sem).wait()
    pltpu.async_copy(trl_hbm.at[idx1_v], b_v, sem).wait()
    for i in range(RPW):
        for j in range((T * TD) // L):
            sl = pl.ds(j * L, L)
            a_v[i, sl] = a_v[i, sl] * b_v[i, sl]
    pltpu.sync_copy(a_v, hrt_out.at[pl.ds(base, RPW)])

    pltpu.sync_copy(wrelT_hbm, wt_v)
    for t in range(T):
        wtmp_v[pl.ds(t * L, L)] = plsc.load_gather(wt_v, [i1 + t * 240])
    for t in range(T):
        pltpu.sync_copy(wtmp_v.at[pl.ds(t * L, L)],
                        wrel_out.at[t, pl.ds(base, RPW)])


def _run_gather(pos0, pos1, ent_emb, rel_emb, te_flat, trl_flat, wrelT_pad):
    mesh = plsc.VectorSubcoreMesh(core_axis_name="c", subcore_axis_name="s")
    f = pl.kernel(
        _gather_body,
        out_type=(
            jax.ShapeDtypeStruct((B, D), jnp.float32),
            jax.ShapeDtypeStruct((B, T * TD), jnp.float32),
            jax.ShapeDtypeStruct((T, B), jnp.float32),
        ),
        mesh=mesh,
        scratch_types=[
            pltpu.VMEM((RPW,), jnp.int32),
            pltpu.VMEM((RPW,), jnp.int32),
            pltpu.VMEM((RPW, D), jnp.float32),
            pltpu.VMEM((RPW, D), jnp.float32),
            pltpu.VMEM((RPW, T * TD), jnp.float32),
            pltpu.VMEM((RPW, T * TD), jnp.float32),
            pltpu.VMEM((T * 240,), jnp.float32),
            pltpu.VMEM((T * L,), jnp.float32),
            pltpu.SemaphoreType.DMA,
        ],
        compiler_params=_SC_PARAMS,
        name="mulde_gather_sc",
    )
    return f(pos0, pos1, ent_emb, rel_emb, te_flat, trl_flat, wrelT_pad)


def _mm_body(hr_ref, ent_ref, score_ref, mx_ref, sm_ref, sq_ref):
    i = pl.program_id(0)
    s = lax.dot_general(hr_ref[...], ent_ref[...],
                        (((1,), (1,)), ((), ())),
                        preferred_element_type=jnp.float32)
    col = i * ET + lax.broadcasted_iota(jnp.int32, (1, ET), 1)
    valid = col < NENT
    s = jnp.where(valid, s, PAD_VAL)
    score_ref[...] = s
    s_z = jnp.where(valid, s, 0.0)
    mx_ref[0, 0, :] = jnp.max(s, axis=1)
    sm_ref[0, 0, :] = jnp.sum(s_z, axis=1)
    sq_ref[0, 0, :] = jnp.sum(s_z * s_z, axis=1)


def _run_matmul(hr, ent_pad):
    return pl.pallas_call(
        _mm_body,
        grid=(NT,),
        in_specs=[
            pl.BlockSpec((B, D), lambda i: (0, 0)),
            pl.BlockSpec((ET, D), lambda i: (i, 0)),
        ],
        out_specs=[
            pl.BlockSpec((B, ET), lambda i: (0, i)),
            pl.BlockSpec((1, 1, B), lambda i: (i, 0, 0)),
            pl.BlockSpec((1, 1, B), lambda i: (i, 0, 0)),
            pl.BlockSpec((1, 1, B), lambda i: (i, 0, 0)),
        ],
        out_shape=[
            jax.ShapeDtypeStruct((B, NENT_PAD), jnp.float32),
            jax.ShapeDtypeStruct((NT, 1, B), jnp.float32),
            jax.ShapeDtypeStruct((NT, 1, B), jnp.float32),
            jax.ShapeDtypeStruct((NT, 1, B), jnp.float32),
        ],
        name="mulde_score_mm",
    )(hr, ent_pad)


UNR = 8
CBUF = CAP + L
BIGI = 1 << 30
NEC = K
EQB = L * NEC


def _count_row_acc(row_v, tau):
    tb = _bc_f(tau)

    def body(i, carry):
        a0, a1 = carry
        for u in range(UNR):
            x = row_v[pl.ds((i * UNR + u) * L, L)]
            c = jnp.where(x > tb, 1, 0)
            if u % 2 == 0:
                a0 = a0 + c
            else:
                a1 = a1 + c
        return a0, a1

    z = jnp.zeros((L,), jnp.int32)
    a0, a1 = lax.fori_loop(0, NVR // UNR, body, (z, z))
    return a0 + a1


def _count_row(row_v, tau):
    return jnp.sum(_count_row_acc(row_v, tau))


def _collect_row(row_v, tau, acc, cval_v, cidx_v):
    tb = _bc_f(tau)
    offs = plsc.cumsum(acc) - acc

    def body(i, st):
        pos, gid = st
        for u in range(UNR):
            x = row_v[pl.ds((i * UNR + u) * L, L)]
            m = x > tb
            plsc.store_scatter(cval_v, [pos], x, mask=m)
            plsc.store_scatter(cidx_v, [pos], gid, mask=m)
            pos = pos + jnp.where(m, 1, 0)
            gid = gid + L
        return pos, gid

    lax.fori_loop(0, NVR // UNR, body, (offs, _lanes()))


def _topk_body(score_hbm, tau_hbm, mx_hbm, pos2_hbm, neg_hbm,
               sval_out, sidx_out, pos_out, neg_out,
               rowA_v, tau_v, mx_v, p2_v, cval_v, cidx_v, eqi_v,
               fval_v, fidx_v, negi_v, nego_v, pv_v):
    wid = lax.axis_index("s") * NC + lax.axis_index("c")
    base = wid * RPW
    lanes = _lanes()
    lane0 = lanes == 0
    nbv = _bc_f(jnp.float32(NEG_BIG))
    bigv = _bc_i(jnp.int32(BIGI))

    pltpu.sync_copy(tau_hbm.at[pl.ds(base, RPW)], tau_v)
    pltpu.sync_copy(mx_hbm.at[pl.ds(base, RPW)], mx_v)
    pltpu.sync_copy(pos2_hbm.at[pl.ds(base, RPW)], p2_v)

    def process(row_v, rl, posvec):
        row = base + rl
        tau0 = _lane_f(tau_v[...], rl)
        mxs = _lane_f(mx_v[...], rl)
        k_mx = jnp.max(_key_vec(_bc_f(mxs)))

        acc0 = _count_row_acc(row_v, tau0)
        cnt0 = jnp.sum(acc0)
        ok0 = (cnt0 >= K) & (cnt0 <= CAP)

        def needs_fix():
            k_tau = jnp.max(_key_vec(_bc_f(tau0)))
            lo0 = jnp.where(cnt0 < K, jnp.int32(_KLO), k_tau)
            hi0 = jnp.where(cnt0 < K, k_tau, k_mx)
            ch0 = jnp.where(cnt0 < K, cnt0, jnp.int32(0))

            def cond(st):
                klo, khi, cnt_hi, acc_tau, accepted = st
                return (accepted == 0) & (_avg_i32(klo, khi) != klo)

            def body(st):
                klo, khi, cnt_hi, acc_tau, accepted = st
                kmid = _avg_i32(klo, khi)
                tm = _unmap_scalar(kmid)
                c = _count_row(row_v, tm)
                acc = (c >= K) & (c <= CAP)
                ge = c >= K
                return (jnp.where(ge, kmid, klo),
                        jnp.where(ge, khi, kmid),
                        jnp.where(ge, cnt_hi, c),
                        jnp.where(acc, tm, acc_tau),
                        jnp.where(acc, jnp.int32(1), jnp.int32(0)))

            klo, khi, cnt_hi, acc_tau, accepted = lax.while_loop(
                cond, body,
                (lo0, hi0, ch0, jnp.float32(0), jnp.int32(0)))
            return accepted, acc_tau, khi, cnt_hi

        def no_fix():
            return (jnp.int32(1), tau0, jnp.int32(0), jnp.int32(0))

        accepted, tau_f, khi_dg, cgt_dg = lax.cond(ok0, no_fix, needs_fix)

        def normal():
            accf = lax.cond(ok0, lambda: acc0,
                            lambda: _count_row_acc(row_v, tau_f))
            _collect_row(row_v, tau_f, accf, cval_v, cidx_v)
            return jnp.sum(accf)

        def degen():
            v64 = _unmap_scalar(khi_dg)
            vb = _bc_f(v64)
            c_gt = cgt_dg
            accg = _count_row_acc(row_v, v64)
            offs_g = plsc.cumsum(accg) - accg
            for q in range(EQB // L):
                eqi_v[pl.ds(q * L, L)] = bigv
            lim = lanes * NEC + NEC

            def bodyB(i, st):
                posg, pose, gid = st
                for u in range(UNR):
                    x = row_v[pl.ds((i * UNR + u) * L, L)]
                    mg = x > vb
                    plsc.store_scatter(cval_v, [posg], x, mask=mg)
                    plsc.store_scatter(cidx_v, [posg], gid, mask=mg)
                    posg = posg + jnp.where(mg, 1, 0)
                    me = x == vb
                    me2 = me & (pose < lim)
                    plsc.store_scatter(eqi_v, [pose], gid, mask=me2)
                    pose = pose + jnp.where(me, 1, 0)
                    gid = gid + L
                return posg, pose, gid

            lax.fori_loop(0, NVR // UNR, bodyB,
                          (offs_g, lanes * NEC, lanes))

            def sel(j, _):
                def mn_body(q, mv):
                    return jnp.minimum(mv, eqi_v[pl.ds(q * L, L)])

                mn = jnp.min(lax.fori_loop(0, EQB // L, mn_body, bigv))
                mnb = _bc_i(mn)

                def cl_body(q, _):
                    e = eqi_v[pl.ds(q * L, L)]
                    eqi_v[pl.ds(q * L, L)] = jnp.where(e == mnb, bigv, e)
                    return 0

                lax.fori_loop(0, EQB // L, cl_body, 0)
                plsc.store_scatter(cval_v, [_bc_i(c_gt + j)], vb, mask=lane0)
                plsc.store_scatter(cidx_v, [_bc_i(c_gt + j)], mnb, mask=lane0)
                return 0

            lax.fori_loop(0, K - c_gt, sel, 0)
            return jnp.int32(K)

        cnt_f = lax.cond(accepted == 1, normal, degen)

        vs = []
        js = []
        for q in range(CBUF // L):
            p = q * L + lanes
            valid = p < cnt_f
            vs.append(jnp.where(valid, cval_v[pl.ds(q * L, L)], nbv))
            js.append(jnp.where(valid, cidx_v[pl.ds(q * L, L)], bigv))
        nq = CBUF // L

        def ext_body(k, st):
            vv = list(st[:nq])
            jj = list(st[nq:])
            mx8 = vv[0]
            for q in range(1, nq):
                mx8 = jnp.maximum(mx8, vv[q])
            mv = jnp.max(mx8)
            mb = _bc_f(mv)
            mn8 = jnp.where(vv[0] == mb, jj[0], bigv)
            for q in range(1, nq):
                mn8 = jnp.minimum(mn8, jnp.where(vv[q] == mb, jj[q], bigv))
            iv = jnp.min(mn8)
            ivb = _bc_i(iv)
            plsc.store_scatter(fval_v, [_bc_i(k)], mb, mask=lane0)
            plsc.store_scatter(fidx_v, [_bc_i(k)], ivb, mask=lane0)
            for q in range(nq):
                vv[q] = jnp.where((vv[q] == mb) & (jj[q] == ivb), nbv, vv[q])
            return tuple(vv) + tuple(jj)

        lax.fori_loop(0, K, ext_body, tuple(vs) + tuple(js))

        pltpu.sync_copy(fval_v, sval_out.at[row])
        pltpu.sync_copy(fidx_v, sidx_out.at[row])

        pltpu.sync_copy(neg_hbm.at[row], negi_v)
        for j in range(NEG // L):
            idx = negi_v[pl.ds(j * L, L)]
            nego_v[pl.ds(j * L, L)] = plsc.load_gather(row_v, [idx])
        pltpu.sync_copy(nego_v, neg_out.at[row])
        p2 = _lane_i(p2_v[...], rl)
        pos_s = jnp.max(plsc.load_gather(row_v, [_bc_i(p2)]))
        return jnp.where(lanes == rl, pos_s, posvec)

    def row_body(rl, posvec):
        pltpu.sync_copy(score_hbm.at[base + rl], rowA_v)
        return process(rowA_v, rl, posvec)

    posvec = lax.fori_loop(0, RPW, row_body, jnp.zeros((L,), jnp.float32))
    pv_v[...] = posvec
    pltpu.sync_copy(pv_v, pos_out.at[pl.ds(base, RPW)])


def _run_topk(score, tau0, mx, pos2, negative):
    mesh = plsc.VectorSubcoreMesh(core_axis_name="c", subcore_axis_name="s")
    f = pl.kernel(
        _topk_body,
        out_type=(
            jax.ShapeDtypeStruct((B, K), jnp.float32),
            jax.ShapeDtypeStruct((B, K), jnp.int32),
            jax.ShapeDtypeStruct((B,), jnp.float32),
            jax.ShapeDtypeStruct((B, NEG), jnp.float32),
        ),
        mesh=mesh,
        scratch_types=[
            pltpu.VMEM((NENT_PAD,), jnp.float32),
            pltpu.VMEM((RPW,), jnp.float32),
            pltpu.VMEM((RPW,), jnp.float32),
            pltpu.VMEM((RPW,), jnp.int32),
            pltpu.VMEM((CBUF,), jnp.float32),
            pltpu.VMEM((CBUF,), jnp.int32),
            pltpu.VMEM((EQB,), jnp.int32),
            pltpu.VMEM((K,), jnp.float32),
            pltpu.VMEM((K,), jnp.int32),
            pltpu.VMEM((NEG,), jnp.int32),
            pltpu.VMEM((NEG,), jnp.float32),
            pltpu.VMEM((RPW,), jnp.float32),
        ],
        compiler_params=_SC_PARAMS,
        name="mulde_topk_sc",
    )
    return f(score, tau0, mx, pos2, negative)


def _log_sigmoid(x):
    return jnp.minimum(x, 0.0) - jnp.log(1.0 + jnp.exp(-jnp.abs(x)))


def _c1_body(pos_ref, neg_ref, a_ref, b_ref):
    a_ref[...] = _log_sigmoid(pos_ref[...])
    b_ref[...] = jnp.mean(_log_sigmoid(-neg_ref[...]), axis=1)


def _run_hardloss(pos_score, neg_score):
    return pl.pallas_call(
        _c1_body,
        out_shape=[
            jax.ShapeDtypeStruct((B,), jnp.float32),
            jax.ShapeDtypeStruct((B,), jnp.float32),
        ],
        name="mulde_hardloss",
    )(pos_score, neg_score)


def kernel(positive, negative, subsampling_weight, epoch, ent_emb, rel_emb,
           Wrel, t_ent_emb, t_rel_emb):
    pos0 = positive[:, 0].astype(jnp.int32)
    pos1 = positive[:, 1].astype(jnp.int32)
    pos2 = positive[:, 2].astype(jnp.int32)
    negative = negative.astype(jnp.int32)
    w = subsampling_weight

    te_flat = t_ent_emb.transpose(1, 0, 2).reshape(NENT, T * TD)
    trl_flat = t_rel_emb.transpose(1, 0, 2).reshape(NREL, T * TD)
    wrelT_pad = jnp.pad(Wrel.T, ((0, 0), (0, 240 - NREL))).reshape(T * 240)
    ent_pad = jnp.pad(ent_emb, ((0, NENT_PAD - NENT), (0, 0)))

    hr, hrt, wrelT = _run_gather(pos0, pos1, ent_emb, rel_emb, te_flat,
                                 trl_flat, wrelT_pad)

    score, mx_t, sm_t, sq_t = _run_matmul(hr, ent_pad)

    mx = jnp.max(mx_t[:, 0, :], axis=0)
    mu = jnp.sum(sm_t[:, 0, :], axis=0) / NENT
    var = jnp.maximum(jnp.sum(sq_t[:, 0, :], axis=0) / NENT - mu * mu, 0.0)
    tau0 = mu + 2.92 * jnp.sqrt(var)

    s_score, s_idx, pos_score, neg_score = _run_topk(
        score, tau0, mx, pos2, negative)

    a_ls, b_ls = _run_hardloss(pos_score, neg_score)

    tt_rows = te_flat[s_idx.reshape(-1)]
    hrt4 = hrt.reshape(B, T, TD).transpose(1, 0, 2)
    tt4 = tt_rows.reshape(B, K, T, TD).transpose(2, 0, 1, 3)
    t_score = jnp.sum(hrt4[:, :, None, :] * tt4, axis=-1)

    sw = jnp.sum(w)
    pos_loss = -jnp.sum(a_ls * w) / sw
    neg_loss = -jnp.sum(b_ls * w) / sw
    hard_loss = (pos_loss + neg_loss) / 2.0

    wrel = wrelT.T
    w_soft = jax.nn.softmax(wrel, axis=1)
    agg_t = jnp.einsum('bt,tbk->bk', w_soft, t_score)
    temp = 2.0
    log_p_s = jax.nn.log_softmax(s_score / temp, axis=1)
    p_t = jax.nn.softmax(agg_t / temp, axis=1)
    log_p_t = jax.nn.log_softmax(agg_t / temp, axis=1)
    kl = jnp.sum(p_t * (log_p_t - log_p_s), axis=1)
    soft_loss = jnp.sum(kl * w) / sw
    alpha = jnp.minimum(1.0, jnp.asarray(epoch, jnp.float32) / 10.0)
    loss = hard_loss + alpha * soft_loss
    return (loss, hard_loss, soft_loss)
